# baseline (speedup 1.0000x reference)
import functools

import jax
import jax.numpy as jnp
from jax import lax
from jax.experimental import pallas as pl
from jax.experimental.pallas import tpu as pltpu
from jax.experimental.pallas import tpu_sc as plsc

_N = 10000
_E = 320000
_D = 128

_NC = 2
_NS = 16
_CH = 125
_EROWS = _E // _CH
_ECH = _EROWS // (_NC * _NS)
_DCH = _EROWS // _NS
_NP = 10240
_RPS = _NP // _NS
_ZR = 128
_ZW = 80


def _mesh():
    return plsc.VectorSubcoreMesh(
        core_axis_name="c", subcore_axis_name="s",
        num_cores=_NC, num_subcores=_NS)


_DW = 128


def _sc_degrees_body(e2d_hbm, out_hbm, acc_sh, idx_v, ones_v):
    cid = lax.axis_index("c")
    sid = lax.axis_index("s")
    for i in range(_CH):
        for j in range(_DW // 16):
            ones_v[i, pl.ds(j * 16, 16)] = jnp.zeros((16,), jnp.float32)
    for k in range(_RPS // _ZW):
        pltpu.sync_copy(ones_v.at[pl.ds(0, _ZW)],
                        acc_sh.at[pl.ds(sid * _RPS + k * _ZW, _ZW)])
    for i in range(_CH):
        for j in range(_DW // 16):
            ones_v[i, pl.ds(j * 16, 16)] = jnp.full((16,), 1.0, jnp.float32)
    plsc.subcore_barrier()
    pltpu.sync_copy(e2d_hbm.at[pl.ds(cid * _EROWS + sid * _DCH, _DCH)], idx_v)

    def step(g, carry):
        pltpu.sync_copy(ones_v, acc_sh.at[idx_v.at[g]], add=True)
        return carry

    lax.fori_loop(0, _DCH, step, 0)
    plsc.subcore_barrier()
    for k in range(_RPS // _ZR):
        r0 = sid * _RPS + k * _ZR
        pltpu.sync_copy(acc_sh.at[pl.ds(r0, _ZR)], out_hbm.at[cid, pl.ds(r0, _ZR)])


def _sc_degrees(e2d):
    f = pl.kernel(
        _sc_degrees_body,
        out_type=jax.ShapeDtypeStruct((_NC, _NP, _DW), jnp.float32),
        mesh=_mesh(),
        scratch_types=[
            pltpu.VMEM_SHARED((_NP, _DW), jnp.float32),
            pltpu.VMEM((_DCH, _CH), jnp.int32),
            pltpu.VMEM((_CH, _DW), jnp.float32),
        ],
    )
    return f(e2d)


_HA = 64
_HB = _CH - _HA


def _sc_edge_body(h_hbm, e2d_hbm, out_hbm, acc_sh, sidx_v, didx_v, buf0, buf1,
                  sg0, sg1, ss0, ss1):
    cid = lax.axis_index("c")
    sid = lax.axis_index("s")
    for i in range(_HA):
        for j in range(_D // 16):
            buf0[i, pl.ds(j * 16, 16)] = jnp.zeros((16,), jnp.float32)
    for k in range(_RPS // _HA):
        pltpu.sync_copy(buf0, acc_sh.at[pl.ds(sid * _RPS + k * _HA, _HA)])
    plsc.subcore_barrier()
    w = cid * _NS + sid
    pltpu.sync_copy(e2d_hbm.at[pl.ds(w * _ECH, _ECH)], sidx_v)
    pltpu.sync_copy(e2d_hbm.at[pl.ds(_EROWS + w * _ECH, _ECH)], didx_v)

    def ga(g):
        return pltpu.async_copy(
            h_hbm.at[sidx_v.at[g, pl.ds(0, _HA)]], buf0, sg0)

    def gb(g):
        return pltpu.async_copy(
            h_hbm.at[sidx_v.at[g, pl.ds(_HA, _HB)]], buf1.at[pl.ds(0, _HB)], sg1)

    ga(0)
    gb(0)

    def step(g, carry):
        pltpu.make_async_copy(
            h_hbm.at[sidx_v.at[g, pl.ds(0, _HA)]], buf0, sg0).wait()
        sca = pltpu.async_copy(
            buf0, acc_sh.at[didx_v.at[g, pl.ds(0, _HA)]], ss0, add=True)
        pltpu.make_async_copy(
            h_hbm.at[sidx_v.at[g, pl.ds(_HA, _HB)]], buf1.at[pl.ds(0, _HB)],
            sg1).wait()
        scb = pltpu.async_copy(
            buf1.at[pl.ds(0, _HB)], acc_sh.at[didx_v.at[g, pl.ds(_HA, _HB)]],
            ss1, add=True)
        sca.wait()

        @pl.when(g + 1 < _ECH)
        def _():
            ga(g + 1)
        scb.wait()

        @pl.when(g + 1 < _ECH)
        def _():
            gb(g + 1)
        return carry

    lax.fori_loop(0, _ECH, step, 0)
    plsc.subcore_barrier()
    for k in range(_RPS // _ZR):
        r0 = sid * _RPS + k * _ZR
        pltpu.sync_copy(acc_sh.at[pl.ds(r0, _ZR)], out_hbm.at[cid, pl.ds(r0, _ZR)])


def _sc_edge_agg(h, e2d):
    f = pl.kernel(
        _sc_edge_body,
        out_type=jax.ShapeDtypeStruct((_NC, _NP, _D), jnp.float32),
        mesh=_mesh(),
        scratch_types=[
            pltpu.VMEM_SHARED((_NP, _D), jnp.float32),
            pltpu.VMEM((_ECH, _CH), jnp.int32),
            pltpu.VMEM((_ECH, _CH), jnp.int32),
            pltpu.VMEM((_HA, _D), jnp.float32),
            pltpu.VMEM((_HA, _D), jnp.float32),
            pltpu.SemaphoreType.DMA,
            pltpu.SemaphoreType.DMA,
            pltpu.SemaphoreType.DMA,
            pltpu.SemaphoreType.DMA,
        ],
    )
    return f(h, e2d)


_BR = 2000


def _norm_col(deg_blk):
    return lax.rsqrt(jnp.maximum(deg_blk[:, :1], 1.0))


def _tc_pre_body(x_ref, dout_ref, g_ref, b_ref, w_ref, o_ref):
    x = x_ref[...]
    m = jnp.mean(x, axis=-1, keepdims=True)
    xc = x - m
    v = jnp.mean(xc * xc, axis=-1, keepdims=True)
    h = xc * lax.rsqrt(v + 1e-5) * g_ref[...] + b_ref[...]
    ns = _norm_col(dout_ref[...])
    o_ref[...] = jnp.dot(h, w_ref[...], preferred_element_type=jnp.float32) * ns


def _tc_pre(feats, deg_out, ln_g, ln_b, W):
    grid = _N // _BR
    return pl.pallas_call(
        _tc_pre_body,
        grid=(grid,),
        in_specs=[
            pl.BlockSpec((_BR, _D), lambda i: (i, 0)),
            pl.BlockSpec((_BR, 8), lambda i: (i, 0)),
            pl.BlockSpec((1, _D), lambda i: (0, 0)),
            pl.BlockSpec((1, _D), lambda i: (0, 0)),
            pl.BlockSpec((_D, _D), lambda i: (0, 0)),
        ],
        out_specs=pl.BlockSpec((_BR, _D), lambda i: (i, 0)),
        out_shape=jax.ShapeDtypeStruct((_N, _D), jnp.float32),
    )(feats, deg_out, ln_g.reshape(1, _D), ln_b.reshape(1, _D), W)


def _tc_mid_body(p_ref, din_ref, dout_ref, b0_ref, g_ref, b_ref, w_ref, o_ref):
    a = p_ref[0] + p_ref[1]
    nd = _norm_col(din_ref[...])
    h = jnp.maximum(a * nd + b0_ref[...], 0.0)
    m = jnp.mean(h, axis=-1, keepdims=True)
    hc = h - m
    v = jnp.mean(hc * hc, axis=-1, keepdims=True)
    h = hc * lax.rsqrt(v + 1e-5) * g_ref[...] + b_ref[...]
    ns = _norm_col(dout_ref[...])
    o_ref[...] = jnp.dot(h, w_ref[...], preferred_element_type=jnp.float32) * ns


def _tc_mid(parts, deg_in, deg_out, b0, ln_g, ln_b, W):
    grid = _N // _BR
    return pl.pallas_call(
        _tc_mid_body,
        grid=(grid,),
        in_specs=[
            pl.BlockSpec((_NC, _BR, _D), lambda i: (0, i, 0)),
            pl.BlockSpec((_BR, 8), lambda i: (i, 0)),
            pl.BlockSpec((_BR, 8), lambda i: (i, 0)),
            pl.BlockSpec((1, _D), lambda i: (0, 0)),
            pl.BlockSpec((1, _D), lambda i: (0, 0)),
            pl.BlockSpec((1, _D), lambda i: (0, 0)),
            pl.BlockSpec((_D, _D), lambda i: (0, 0)),
        ],
        out_specs=pl.BlockSpec((_BR, _D), lambda i: (i, 0)),
        out_shape=jax.ShapeDtypeStruct((_N, _D), jnp.float32),
    )(parts, deg_in, deg_out, b0.reshape(1, _D), ln_g.reshape(1, _D),
      ln_b.reshape(1, _D), W)


def _tc_post_body(p_ref, din_ref, b1_ref, o_ref):
    a = p_ref[0] + p_ref[1]
    nd = _norm_col(din_ref[...])
    o_ref[...] = a * nd + b1_ref[...]


def _tc_post(parts, deg_in, b1):
    grid = _N // _BR
    return pl.pallas_call(
        _tc_post_body,
        grid=(grid,),
        in_specs=[
            pl.BlockSpec((_NC, _BR, _D), lambda i: (0, i, 0)),
            pl.BlockSpec((_BR, 8), lambda i: (i, 0)),
            pl.BlockSpec((1, _D), lambda i: (0, 0)),
        ],
        out_specs=pl.BlockSpec((_BR, _D), lambda i: (i, 0)),
        out_shape=jax.ShapeDtypeStruct((_N, _D), jnp.float32),
    )(parts, deg_in, b1.reshape(1, _D))


def kernel(feats, edge_index, ln0_g, ln0_b, ln1_g, ln1_b, W0, b0, W1, b1):
    e2d = edge_index.reshape(2 * _EROWS, _CH)
    degs = _sc_degrees(e2d)
    deg_out = degs[0, :, :8]
    deg_in = degs[1, :, :8]
    h0 = _tc_pre(feats, deg_out, ln0_g, ln0_b, W0)
    p0 = _sc_edge_agg(h0, e2d)
    h1 = _tc_mid(p0, deg_in, deg_out, b0, ln1_g, ln1_b, W1)
    p1 = _sc_edge_agg(h1, e2d)
    return _tc_post(p1, deg_in, b1)


# speedup vs baseline: 1.1726x; 1.1726x over previous
import functools

import jax
import jax.numpy as jnp
from jax import lax
from jax.experimental import pallas as pl
from jax.experimental.pallas import tpu as pltpu
from jax.experimental.pallas import tpu_sc as plsc

_N = 10000
_E = 320000
_D = 128

_NC = 2
_NS = 16
_CH = 125
_EROWS = _E // _CH
_ECH = _EROWS // (_NC * _NS)
_DCH = _EROWS // _NS
_NP = 10240
_RPS = _NP // _NS
_ZR = 128
_ZW = 80


def _mesh():
    return plsc.VectorSubcoreMesh(
        core_axis_name="c", subcore_axis_name="s",
        num_cores=_NC, num_subcores=_NS)


_DW = 128


def _sc_degrees_body(e2d_hbm, out_hbm, acc_sh, idx_v, ones_v):
    cid = lax.axis_index("c")
    sid = lax.axis_index("s")
    for i in range(_CH):
        for j in range(_DW // 16):
            ones_v[i, pl.ds(j * 16, 16)] = jnp.zeros((16,), jnp.float32)
    for k in range(_RPS // _ZW):
        pltpu.sync_copy(ones_v.at[pl.ds(0, _ZW)],
                        acc_sh.at[pl.ds(sid * _RPS + k * _ZW, _ZW)])
    for i in range(_CH):
        for j in range(_DW // 16):
            ones_v[i, pl.ds(j * 16, 16)] = jnp.full((16,), 1.0, jnp.float32)
    plsc.subcore_barrier()
    pltpu.sync_copy(e2d_hbm.at[pl.ds(cid * _EROWS + sid * _DCH, _DCH)], idx_v)

    def step(g, carry):
        pltpu.sync_copy(ones_v, acc_sh.at[idx_v.at[g]], add=True)
        return carry

    lax.fori_loop(0, _DCH, step, 0)
    plsc.subcore_barrier()
    for k in range(_RPS // _ZR):
        r0 = sid * _RPS + k * _ZR
        pltpu.sync_copy(acc_sh.at[pl.ds(r0, _ZR)], out_hbm.at[cid, pl.ds(r0, _ZR)])


def _sc_degrees(e2d):
    f = pl.kernel(
        _sc_degrees_body,
        out_type=jax.ShapeDtypeStruct((_NC, _NP, _DW), jnp.float32),
        mesh=_mesh(),
        scratch_types=[
            pltpu.VMEM_SHARED((_NP, _DW), jnp.float32),
            pltpu.VMEM((_DCH, _CH), jnp.int32),
            pltpu.VMEM((_CH, _DW), jnp.float32),
        ],
    )
    return f(e2d)


_HA = 64
_HB = _CH - _HA


def _sc_edge_body(h_hbm, e2d_hbm, out_hbm, acc_sh, sidx_v, didx_v, buf0, buf1,
                  sg0, sg1):
    cid = lax.axis_index("c")
    sid = lax.axis_index("s")
    for i in range(_HA):
        for j in range(_D // 16):
            buf0[i, pl.ds(j * 16, 16)] = jnp.zeros((16,), jnp.float32)
    for k in range(_RPS // _HA):
        pltpu.sync_copy(buf0, acc_sh.at[pl.ds(sid * _RPS + k * _HA, _HA)])
    plsc.subcore_barrier()
    w = cid * _NS + sid
    pltpu.sync_copy(e2d_hbm.at[pl.ds(w * _ECH, _ECH)], sidx_v)
    pltpu.sync_copy(e2d_hbm.at[pl.ds(_EROWS + w * _ECH, _ECH)], didx_v)

    def ga(g):
        return pltpu.async_copy(
            h_hbm.at[sidx_v.at[g, pl.ds(0, _HA)]], buf0, sg0)

    def gb(g):
        return pltpu.async_copy(
            h_hbm.at[sidx_v.at[g, pl.ds(_HA, _HB)]], buf1.at[pl.ds(0, _HB)], sg1)

    ga(0)
    gb(0)

    def step(g, carry):
        pltpu.make_async_copy(
            h_hbm.at[sidx_v.at[g, pl.ds(0, _HA)]], buf0, sg0).wait()
        pltpu.sync_copy(buf0, acc_sh.at[didx_v.at[g, pl.ds(0, _HA)]], add=True)

        @pl.when(g + 1 < _ECH)
        def _():
            ga(g + 1)
        pltpu.make_async_copy(
            h_hbm.at[sidx_v.at[g, pl.ds(_HA, _HB)]], buf1.at[pl.ds(0, _HB)],
            sg1).wait()
        pltpu.sync_copy(buf1.at[pl.ds(0, _HB)],
                        acc_sh.at[didx_v.at[g, pl.ds(_HA, _HB)]], add=True)

        @pl.when(g + 1 < _ECH)
        def _():
            gb(g + 1)
        return carry

    lax.fori_loop(0, _ECH, step, 0)
    plsc.subcore_barrier()
    for k in range(_RPS // _ZR):
        r0 = sid * _RPS + k * _ZR
        pltpu.sync_copy(acc_sh.at[pl.ds(r0, _ZR)], out_hbm.at[cid, pl.ds(r0, _ZR)])


def _sc_edge_agg(h, e2d):
    f = pl.kernel(
        _sc_edge_body,
        out_type=jax.ShapeDtypeStruct((_NC, _NP, _D), jnp.float32),
        mesh=_mesh(),
        scratch_types=[
            pltpu.VMEM_SHARED((_NP, _D), jnp.float32),
            pltpu.VMEM((_ECH, _CH), jnp.int32),
            pltpu.VMEM((_ECH, _CH), jnp.int32),
            pltpu.VMEM((_HA, _D), jnp.float32),
            pltpu.VMEM((_HA, _D), jnp.float32),
            pltpu.SemaphoreType.DMA,
            pltpu.SemaphoreType.DMA,
        ],
    )
    return f(h, e2d)


_BR = 2000


def _norm_col(deg_blk):
    return lax.rsqrt(jnp.maximum(deg_blk[:, :1], 1.0))


def _tc_pre_body(x_ref, dout_ref, g_ref, b_ref, w_ref, o_ref):
    x = x_ref[...]
    m = jnp.mean(x, axis=-1, keepdims=True)
    xc = x - m
    v = jnp.mean(xc * xc, axis=-1, keepdims=True)
    h = xc * lax.rsqrt(v + 1e-5) * g_ref[...] + b_ref[...]
    ns = _norm_col(dout_ref[0])
    o_ref[...] = jnp.dot(h, w_ref[...], preferred_element_type=jnp.float32) * ns


def _tc_pre(feats, degs, ln_g, ln_b, W):
    grid = _N // _BR
    return pl.pallas_call(
        _tc_pre_body,
        grid=(grid,),
        in_specs=[
            pl.BlockSpec((_BR, _D), lambda i: (i, 0)),
            pl.BlockSpec((1, _BR, _D), lambda i: (0, i, 0)),
            pl.BlockSpec((1, _D), lambda i: (0, 0)),
            pl.BlockSpec((1, _D), lambda i: (0, 0)),
            pl.BlockSpec((_D, _D), lambda i: (0, 0)),
        ],
        out_specs=pl.BlockSpec((_BR, _D), lambda i: (i, 0)),
        out_shape=jax.ShapeDtypeStruct((_N, _D), jnp.float32),
    )(feats, degs, ln_g.reshape(1, _D), ln_b.reshape(1, _D), W)


def _tc_mid_body(p_ref, d_ref, b0_ref, g_ref, b_ref, w_ref, o_ref):
    a = p_ref[0] + p_ref[1]
    nd = _norm_col(d_ref[1])
    h = jnp.maximum(a * nd + b0_ref[...], 0.0)
    m = jnp.mean(h, axis=-1, keepdims=True)
    hc = h - m
    v = jnp.mean(hc * hc, axis=-1, keepdims=True)
    h = hc * lax.rsqrt(v + 1e-5) * g_ref[...] + b_ref[...]
    ns = _norm_col(d_ref[0])
    o_ref[...] = jnp.dot(h, w_ref[...], preferred_element_type=jnp.float32) * ns


def _tc_mid(parts, degs, b0, ln_g, ln_b, W):
    grid = _N // _BR
    return pl.pallas_call(
        _tc_mid_body,
        grid=(grid,),
        in_specs=[
            pl.BlockSpec((_NC, _BR, _D), lambda i: (0, i, 0)),
            pl.BlockSpec((2, _BR, _D), lambda i: (0, i, 0)),
            pl.BlockSpec((1, _D), lambda i: (0, 0)),
            pl.BlockSpec((1, _D), lambda i: (0, 0)),
            pl.BlockSpec((1, _D), lambda i: (0, 0)),
            pl.BlockSpec((_D, _D), lambda i: (0, 0)),
        ],
        out_specs=pl.BlockSpec((_BR, _D), lambda i: (i, 0)),
        out_shape=jax.ShapeDtypeStruct((_N, _D), jnp.float32),
    )(parts, degs, b0.reshape(1, _D), ln_g.reshape(1, _D),
      ln_b.reshape(1, _D), W)


def _tc_post_body(p_ref, din_ref, b1_ref, o_ref):
    a = p_ref[0] + p_ref[1]
    nd = _norm_col(din_ref[0])
    o_ref[...] = a * nd + b1_ref[...]


def _tc_post(parts, degs, b1):
    grid = _N // _BR
    return pl.pallas_call(
        _tc_post_body,
        grid=(grid,),
        in_specs=[
            pl.BlockSpec((_NC, _BR, _D), lambda i: (0, i, 0)),
            pl.BlockSpec((1, _BR, _D), lambda i: (1, i, 0)),
            pl.BlockSpec((1, _D), lambda i: (0, 0)),
        ],
        out_specs=pl.BlockSpec((_BR, _D), lambda i: (i, 0)),
        out_shape=jax.ShapeDtypeStruct((_N, _D), jnp.float32),
    )(parts, degs, b1.reshape(1, _D))


def kernel(feats, edge_index, ln0_g, ln0_b, ln1_g, ln1_b, W0, b0, W1, b1):
    e2d = edge_index.reshape(2 * _EROWS, _CH)
    degs = _sc_degrees(e2d)
    h0 = _tc_pre(feats, degs, ln0_g, ln0_b, W0)
    p0 = _sc_edge_agg(h0, e2d)
    h1 = _tc_mid(p0, degs, b0, ln1_g, ln1_b, W1)
    p1 = _sc_edge_agg(h1, e2d)
    return _tc_post(p1, degs, b1)
